# baseline (speedup 1.0000x reference)
import functools

import jax
import jax.numpy as jnp
from jax.experimental import pallas as pl
from jax.experimental.pallas import tpu as pltpu

_CH = 256
_CG = 512


def _build_kernel(crow_ref, ccol_ref, chunk_ref, first_ref, last_ref,
                  keys_ref, keys_t_ref, out_ref, acc_ref, *, tm, n, cg):
    t = pl.program_id(0)
    cell = crow_ref[t] * (n // cg) + ccol_ref[t]
    keyv = keys_ref[0]
    keyt = keys_t_ref[0]
    hi = keyv >> 9
    rl_iota = jax.lax.broadcasted_iota(jnp.int32, (tm, 1), 0)
    cl_iota = jax.lax.broadcasted_iota(jnp.int32, (1, cg), 1)
    ohr = (hi == cell * 512 + rl_iota).astype(jnp.bfloat16)
    ohc = ((keyt & 511) == cl_iota).astype(jnp.bfloat16)
    m = jax.lax.dot_general(ohr, ohc, (((1,), (0,)), ((), ())),
                            preferred_element_type=jnp.float32)

    @pl.when(first_ref[t] == 1)
    def _():
        acc_ref[...] = m

    @pl.when(first_ref[t] == 0)
    def _():
        acc_ref[...] = acc_ref[...] + m

    @pl.when(last_ref[t] == 1)
    def _():
        out_ref[...] = acc_ref[...].astype(out_ref.dtype)


def _build_counts(keys2, keys2t, crow, ccol, chunk, first, last,
                  n, tm, nitems, dtype):
    cg = min(_CG, n)
    return pl.pallas_call(
        functools.partial(_build_kernel, tm=tm, n=n, cg=cg),
        out_shape=jax.ShapeDtypeStruct((n, n), dtype),
        grid_spec=pltpu.PrefetchScalarGridSpec(
            num_scalar_prefetch=5,
            grid=(nitems,),
            in_specs=[
                pl.BlockSpec(
                    (1, 1, _CH), lambda t, cr, cc, ch, fr, la: (ch[t], 0, 0)),
                pl.BlockSpec(
                    (1, _CH, 1), lambda t, cr, cc, ch, fr, la: (ch[t], 0, 0)),
            ],
            out_specs=pl.BlockSpec(
                (tm, cg), lambda t, cr, cc, ch, fr, la: (cr[t], cc[t])),
            scratch_shapes=[pltpu.VMEM((tm, cg), jnp.float32)],
        ),
        compiler_params=pltpu.CompilerParams(
            dimension_semantics=("arbitrary",)),
    )(crow, ccol, chunk, first, last, keys2, keys2t)


def _edge_tables(src, dst, n, tm):
    e = src.shape[0]
    nch = -(-e // _CH)
    cg = min(_CG, n)
    ncell = (n // tm) * (n // cg)
    r = dst.astype(jnp.int32)
    c = src.astype(jnp.int32)
    cell = (r // tm) * (n // cg) + (c // cg)
    key = (cell << 18) | ((r % tm) << 9) | (c % cg)
    keys = jnp.sort(key)
    sent = jnp.int32(1 << 28)
    keys_p = jnp.concatenate(
        [keys, jnp.full((nch * _CH - e + _CH,), sent, jnp.int32)])
    keys2 = keys_p.reshape(nch + 1, 1, _CH)
    keys2t = keys_p.reshape(nch + 1, _CH, 1)

    qidx = jnp.arange(nch, dtype=jnp.int32)
    first_cell = keys_p[qidx * _CH] >> 18
    last_cell = keys[jnp.minimum((qidx + 1) * _CH - 1, e - 1)] >> 18
    cells = jnp.arange(ncell, dtype=jnp.int32)
    lo = jnp.searchsorted(last_cell, cells, side='left').astype(jnp.int32)
    hi = jnp.searchsorted(first_cell, cells, side='right').astype(jnp.int32) - 1
    cnt_c = jnp.maximum(hi - lo + 1, 1)
    cum = jnp.concatenate(
        [jnp.zeros((1,), jnp.int32), jnp.cumsum(cnt_c).astype(jnp.int32)])

    nitems = ncell + nch - 1
    tt = jnp.arange(nitems, dtype=jnp.int32)
    cell_t = jnp.clip(
        jnp.searchsorted(cum, tt, side='right').astype(jnp.int32) - 1,
        0, ncell - 1)
    k_t = tt - cum[cell_t]
    valid = k_t <= hi[cell_t] - lo[cell_t]
    chunk_t = jnp.where(valid, lo[cell_t] + k_t, nch).astype(jnp.int32)
    first_t = (k_t == 0).astype(jnp.int32)
    last_t = (k_t == cnt_c[cell_t] - 1).astype(jnp.int32)
    crow_t = (cell_t // (n // cg)).astype(jnp.int32)
    ccol_t = (cell_t % (n // cg)).astype(jnp.int32)
    return keys2, keys2t, crow_t, ccol_t, chunk_t, first_t, last_t, nitems


def _layer_kernel(cnt_ref, xfull_ref, wl_ref, wr_ref, gamma_ref, beta_ref,
                  out_ref, *, tm, out_dim, eps, residual, relu):
    i = pl.program_id(0)
    cnt = cnt_ref[...]
    m = jnp.dot(cnt, xfull_ref[...], preferred_element_type=jnp.float32)
    deg = jnp.sum(cnt, axis=-1, keepdims=True).astype(jnp.float32)
    msg = (m * (1.0 / jnp.maximum(deg, 1.0))).astype(cnt.dtype)
    xt = xfull_ref[pl.ds(i * tm, tm), :]
    h = (jnp.dot(xt, wl_ref[...], preferred_element_type=jnp.float32)
         + jnp.dot(msg, wr_ref[...], preferred_element_type=jnp.float32))

    inv_f = 1.0 / out_dim
    s = jnp.sum(h, axis=-1, keepdims=True)
    ss = jnp.sum(h * h, axis=-1, keepdims=True)
    mean = s * inv_f
    var = ss * inv_f - mean * mean
    y = (h - mean) * jax.lax.rsqrt(var + eps) * gamma_ref[...] + beta_ref[...]
    if residual:
        y = y + xt.astype(jnp.float32)
    if relu:
        y = jnp.maximum(y, 0.0)
    out_ref[...] = y.astype(out_ref.dtype)


def _layer(cnt, x_bf, wl_t, wr_t, gamma, beta, *, out_dim, residual, relu,
           out_dtype, eps=1e-5):
    n, in_p = x_bf.shape
    out_p = wl_t.shape[1]
    tm = 512 if n % 512 == 0 else n
    body = functools.partial(_layer_kernel, tm=tm, out_dim=out_dim, eps=eps,
                             residual=residual, relu=relu)
    return pl.pallas_call(
        body,
        out_shape=jax.ShapeDtypeStruct((n, out_p), out_dtype),
        grid=(n // tm,),
        in_specs=[
            pl.BlockSpec((tm, n), lambda i: (i, 0)),
            pl.BlockSpec((n, in_p), lambda i: (0, 0)),
            pl.BlockSpec((in_p, out_p), lambda i: (0, 0)),
            pl.BlockSpec((in_p, out_p), lambda i: (0, 0)),
            pl.BlockSpec((1, out_p), lambda i: (0, 0)),
            pl.BlockSpec((1, out_p), lambda i: (0, 0)),
        ],
        out_specs=pl.BlockSpec((tm, out_p), lambda i: (i, 0)),
        compiler_params=pltpu.CompilerParams(
            dimension_semantics=("parallel",)),
    )(cnt, x_bf, wl_t, wr_t, gamma, beta)


def kernel(x, edge_index,
           l0_w_l, l0_a_l, l0_b_l, l0_w_r, l0_a_r, l0_b_r, l0_gamma, l0_beta,
           l1_w_l, l1_a_l, l1_b_l, l1_w_r, l1_a_r, l1_b_r, l1_gamma, l1_beta):
    n = x.shape[0]
    scaling = 2.0
    bf = jnp.bfloat16

    wl0 = (l0_w_l.T + scaling * (l0_a_l.T @ l0_b_l.T)).astype(bf)
    wr0 = (l0_w_r.T + scaling * (l0_a_r.T @ l0_b_r.T)).astype(bf)
    wl1 = (l1_w_l.T + scaling * (l1_a_l.T @ l1_b_l.T)).astype(bf)
    wr1 = (l1_w_r.T + scaling * (l1_a_r.T @ l1_b_r.T)).astype(bf)
    g0 = l0_gamma.reshape(1, -1).astype(jnp.float32)
    b0 = l0_beta.reshape(1, -1).astype(jnp.float32)
    g1 = l1_gamma.reshape(1, -1).astype(jnp.float32)
    b1 = l1_beta.reshape(1, -1).astype(jnp.float32)

    src, dst = edge_index[0], edge_index[1]
    tm = 512 if n % 512 == 0 else n
    (keys2, keys2t, crow, ccol, chunk, first, last,
     nitems) = _edge_tables(src, dst, n, tm)
    cnt = _build_counts(keys2, keys2t, crow, ccol, chunk, first, last,
                        n, tm, nitems, bf)

    hid = wl0.shape[1]
    out_d = wl1.shape[1]
    h1 = _layer(cnt, x.astype(bf), wl0, wr0, g0, b0, out_dim=hid,
                residual=True, relu=True, out_dtype=bf)
    out = _layer(cnt, h1, wl1, wr1, g1, b1, out_dim=out_d,
                 residual=False, relu=False, out_dtype=jnp.float32)
    return out


# speedup vs baseline: 1.2064x; 1.2064x over previous
import functools

import jax
import jax.numpy as jnp
from jax.experimental import pallas as pl
from jax.experimental.pallas import tpu as pltpu

_CH = 256
_CG = 512


def _build_kernel(crow_ref, ccol_ref, chunk_ref, first_ref, last_ref,
                  keys_ref, out_ref, *, tm, n, cg):
    t = pl.program_id(0)
    cell = crow_ref[t] * (n // cg) + ccol_ref[t]
    keyv = keys_ref[0]
    hi = keyv >> 9
    cl = keyv & 511
    rl_iota = jax.lax.broadcasted_iota(jnp.int32, (tm, 1), 0)
    cl_iota = jax.lax.broadcasted_iota(jnp.int32, (cg, 1), 0)
    f8 = jnp.float8_e4m3fn
    ohr = (hi == cell * 512 + rl_iota).astype(f8)
    ohc = (cl == cl_iota).astype(f8)
    m = jax.lax.dot_general(ohr, ohc, (((1,), (1,)), ((), ())),
                            preferred_element_type=jnp.float32)

    @pl.when(first_ref[t] == 1)
    def _():
        out_ref[...] = m.astype(out_ref.dtype)

    @pl.when(first_ref[t] == 0)
    def _():
        out_ref[...] = out_ref[...] + m.astype(out_ref.dtype)


def _build_counts(keys2, crow, ccol, chunk, first, last,
                  n, tm, nitems, dtype):
    cg = min(_CG, n)
    return pl.pallas_call(
        functools.partial(_build_kernel, tm=tm, n=n, cg=cg),
        out_shape=jax.ShapeDtypeStruct((n, n), dtype),
        grid_spec=pltpu.PrefetchScalarGridSpec(
            num_scalar_prefetch=5,
            grid=(nitems,),
            in_specs=[
                pl.BlockSpec(
                    (1, 1, _CH), lambda t, cr, cc, ch, fr, la: (ch[t], 0, 0)),
            ],
            out_specs=pl.BlockSpec(
                (tm, cg), lambda t, cr, cc, ch, fr, la: (cr[t], cc[t])),
        ),
        compiler_params=pltpu.CompilerParams(
            dimension_semantics=("arbitrary",)),
    )(crow, ccol, chunk, first, last, keys2)


def _edge_tables(src, dst, n, tm):
    e = src.shape[0]
    nch = -(-e // _CH)
    cg = min(_CG, n)
    ncell = (n // tm) * (n // cg)
    r = dst.astype(jnp.int32)
    c = src.astype(jnp.int32)
    cell = (r // tm) * (n // cg) + (c // cg)
    key = (cell << 18) | ((r % tm) << 9) | (c % cg)
    keys = jnp.sort(key)
    sent = jnp.int32(1 << 28)
    keys_p = jnp.concatenate(
        [keys, jnp.full((nch * _CH - e + _CH,), sent, jnp.int32)])
    keys2 = keys_p.reshape(nch + 1, 1, _CH)

    qidx = jnp.arange(nch, dtype=jnp.int32)
    first_cell = keys_p[qidx * _CH] >> 18
    last_cell = keys[jnp.minimum((qidx + 1) * _CH - 1, e - 1)] >> 18
    cells = jnp.arange(ncell, dtype=jnp.int32)
    lo = jnp.searchsorted(last_cell, cells, side='left').astype(jnp.int32)
    hi = jnp.searchsorted(first_cell, cells, side='right').astype(jnp.int32) - 1
    cnt_c = jnp.maximum(hi - lo + 1, 1)
    cum = jnp.concatenate(
        [jnp.zeros((1,), jnp.int32), jnp.cumsum(cnt_c).astype(jnp.int32)])

    nitems = ncell + nch - 1
    tt = jnp.arange(nitems, dtype=jnp.int32)
    cell_t = jnp.clip(
        jnp.searchsorted(cum, tt, side='right').astype(jnp.int32) - 1,
        0, ncell - 1)
    k_t = tt - cum[cell_t]
    valid = k_t <= hi[cell_t] - lo[cell_t]
    chunk_t = jnp.where(valid, lo[cell_t] + k_t, nch).astype(jnp.int32)
    first_t = (k_t == 0).astype(jnp.int32)
    last_t = (k_t == cnt_c[cell_t] - 1).astype(jnp.int32)
    crow_t = (cell_t // (n // cg)).astype(jnp.int32)
    ccol_t = (cell_t % (n // cg)).astype(jnp.int32)
    return keys2, crow_t, ccol_t, chunk_t, first_t, last_t, nitems


def _layer_kernel(cnt_ref, xfull_ref, wl_ref, wr_ref, gamma_ref, beta_ref,
                  out_ref, *, tm, out_dim, eps, residual, relu):
    i = pl.program_id(0)
    cnt = cnt_ref[...]
    m = jnp.dot(cnt, xfull_ref[...], preferred_element_type=jnp.float32)
    deg = jnp.sum(cnt, axis=-1, keepdims=True).astype(jnp.float32)
    msg = (m * (1.0 / jnp.maximum(deg, 1.0))).astype(cnt.dtype)
    xt = xfull_ref[pl.ds(i * tm, tm), :]
    h = (jnp.dot(xt, wl_ref[...], preferred_element_type=jnp.float32)
         + jnp.dot(msg, wr_ref[...], preferred_element_type=jnp.float32))

    inv_f = 1.0 / out_dim
    s = jnp.sum(h, axis=-1, keepdims=True)
    ss = jnp.sum(h * h, axis=-1, keepdims=True)
    mean = s * inv_f
    var = ss * inv_f - mean * mean
    y = (h - mean) * jax.lax.rsqrt(var + eps) * gamma_ref[...] + beta_ref[...]
    if residual:
        y = y + xt.astype(jnp.float32)
    if relu:
        y = jnp.maximum(y, 0.0)
    out_ref[...] = y.astype(out_ref.dtype)


def _layer(cnt, x_bf, wl_t, wr_t, gamma, beta, *, out_dim, residual, relu,
           out_dtype, eps=1e-5):
    n, in_p = x_bf.shape
    out_p = wl_t.shape[1]
    tm = 512 if n % 512 == 0 else n
    body = functools.partial(_layer_kernel, tm=tm, out_dim=out_dim, eps=eps,
                             residual=residual, relu=relu)
    return pl.pallas_call(
        body,
        out_shape=jax.ShapeDtypeStruct((n, out_p), out_dtype),
        grid=(n // tm,),
        in_specs=[
            pl.BlockSpec((tm, n), lambda i: (i, 0)),
            pl.BlockSpec((n, in_p), lambda i: (0, 0)),
            pl.BlockSpec((in_p, out_p), lambda i: (0, 0)),
            pl.BlockSpec((in_p, out_p), lambda i: (0, 0)),
            pl.BlockSpec((1, out_p), lambda i: (0, 0)),
            pl.BlockSpec((1, out_p), lambda i: (0, 0)),
        ],
        out_specs=pl.BlockSpec((tm, out_p), lambda i: (i, 0)),
        compiler_params=pltpu.CompilerParams(
            dimension_semantics=("parallel",)),
    )(cnt, x_bf, wl_t, wr_t, gamma, beta)


def kernel(x, edge_index,
           l0_w_l, l0_a_l, l0_b_l, l0_w_r, l0_a_r, l0_b_r, l0_gamma, l0_beta,
           l1_w_l, l1_a_l, l1_b_l, l1_w_r, l1_a_r, l1_b_r, l1_gamma, l1_beta):
    n = x.shape[0]
    scaling = 2.0
    bf = jnp.bfloat16

    wl0 = (l0_w_l.T + scaling * (l0_a_l.T @ l0_b_l.T)).astype(bf)
    wr0 = (l0_w_r.T + scaling * (l0_a_r.T @ l0_b_r.T)).astype(bf)
    wl1 = (l1_w_l.T + scaling * (l1_a_l.T @ l1_b_l.T)).astype(bf)
    wr1 = (l1_w_r.T + scaling * (l1_a_r.T @ l1_b_r.T)).astype(bf)
    g0 = l0_gamma.reshape(1, -1).astype(jnp.float32)
    b0 = l0_beta.reshape(1, -1).astype(jnp.float32)
    g1 = l1_gamma.reshape(1, -1).astype(jnp.float32)
    b1 = l1_beta.reshape(1, -1).astype(jnp.float32)

    src, dst = edge_index[0], edge_index[1]
    tm = 512 if n % 512 == 0 else n
    (keys2, crow, ccol, chunk, first, last,
     nitems) = _edge_tables(src, dst, n, tm)
    cnt = _build_counts(keys2, crow, ccol, chunk, first, last,
                        n, tm, nitems, bf)

    hid = wl0.shape[1]
    out_d = wl1.shape[1]
    h1 = _layer(cnt, x.astype(bf), wl0, wr0, g0, b0, out_dim=hid,
                residual=True, relu=True, out_dtype=bf)
    out = _layer(cnt, h1, wl1, wr1, g1, b1, out_dim=out_d,
                 residual=False, relu=False, out_dtype=jnp.float32)
    return out


# speedup vs baseline: 1.3984x; 1.1591x over previous
import functools

import jax
import jax.numpy as jnp
from jax.experimental import pallas as pl
from jax.experimental.pallas import tpu as pltpu

_CH = 512
_CG = 512


def _build_kernel(crow_ref, ccol_ref, chunk_ref, first_ref, last_ref,
                  keys_ref, out_ref, *, tm, n, cg):
    t = pl.program_id(0)
    cell = crow_ref[t] * (n // cg) + ccol_ref[t]
    keyv = keys_ref[0]
    hi = keyv >> 9
    cl = keyv & 511
    rl_iota = jax.lax.broadcasted_iota(jnp.int32, (tm, 1), 0)
    cl_iota = jax.lax.broadcasted_iota(jnp.int32, (cg, 1), 0)
    f8 = jnp.float8_e4m3fn
    ohr = (hi == cell * 512 + rl_iota).astype(f8)
    ohc = (cl == cl_iota).astype(f8)
    m = jax.lax.dot_general(ohr, ohc, (((1,), (1,)), ((), ())),
                            preferred_element_type=jnp.float32)

    @pl.when(first_ref[t] == 1)
    def _():
        out_ref[...] = m.astype(out_ref.dtype)

    @pl.when(first_ref[t] == 0)
    def _():
        out_ref[...] = out_ref[...] + m.astype(out_ref.dtype)


def _build_counts(keys2, crow, ccol, chunk, first, last,
                  n, tm, nitems, dtype):
    cg = min(_CG, n)
    return pl.pallas_call(
        functools.partial(_build_kernel, tm=tm, n=n, cg=cg),
        out_shape=jax.ShapeDtypeStruct((n, n), dtype),
        grid_spec=pltpu.PrefetchScalarGridSpec(
            num_scalar_prefetch=5,
            grid=(nitems,),
            in_specs=[
                pl.BlockSpec(
                    (1, 1, _CH), lambda t, cr, cc, ch, fr, la: (ch[t], 0, 0)),
            ],
            out_specs=pl.BlockSpec(
                (tm, cg), lambda t, cr, cc, ch, fr, la: (cr[t], cc[t])),
        ),
        compiler_params=pltpu.CompilerParams(
            dimension_semantics=("arbitrary",)),
    )(crow, ccol, chunk, first, last, keys2)


def _edge_tables(src, dst, n, tm):
    e = src.shape[0]
    nch = -(-e // _CH)
    cg = min(_CG, n)
    ncell = (n // tm) * (n // cg)
    r = dst.astype(jnp.int32)
    c = src.astype(jnp.int32)
    cell = (r // tm) * (n // cg) + (c // cg)
    key = (cell << 18) | ((r % tm) << 9) | (c % cg)
    keys = jnp.sort(key)
    sent = jnp.int32(1 << 28)
    keys_p = jnp.concatenate(
        [keys, jnp.full((nch * _CH - e + _CH,), sent, jnp.int32)])
    keys2 = keys_p.reshape(nch + 1, 1, _CH)

    qidx = jnp.arange(nch, dtype=jnp.int32)
    first_cell = keys_p[qidx * _CH] >> 18
    last_cell = keys[jnp.minimum((qidx + 1) * _CH - 1, e - 1)] >> 18
    cells = jnp.arange(ncell, dtype=jnp.int32)
    lo = jnp.searchsorted(last_cell, cells, side='left').astype(jnp.int32)
    hi = jnp.searchsorted(first_cell, cells, side='right').astype(jnp.int32) - 1
    cnt_c = jnp.maximum(hi - lo + 1, 1)
    cum = jnp.concatenate(
        [jnp.zeros((1,), jnp.int32), jnp.cumsum(cnt_c).astype(jnp.int32)])

    nitems = ncell + nch - 1
    tt = jnp.arange(nitems, dtype=jnp.int32)
    cell_t = jnp.clip(
        jnp.searchsorted(cum, tt, side='right').astype(jnp.int32) - 1,
        0, ncell - 1)
    k_t = tt - cum[cell_t]
    valid = k_t <= hi[cell_t] - lo[cell_t]
    chunk_t = jnp.where(valid, lo[cell_t] + k_t, nch).astype(jnp.int32)
    first_t = (k_t == 0).astype(jnp.int32)
    last_t = (k_t == cnt_c[cell_t] - 1).astype(jnp.int32)
    crow_t = (cell_t // (n // cg)).astype(jnp.int32)
    ccol_t = (cell_t % (n // cg)).astype(jnp.int32)
    return keys2, crow_t, ccol_t, chunk_t, first_t, last_t, nitems


def _layer_kernel(cnt_ref, xfull_ref, wl_ref, wr_ref, gamma_ref, beta_ref,
                  out_ref, *, tm, out_dim, eps, residual, relu):
    i = pl.program_id(0)
    cnt = cnt_ref[...]
    m = jnp.dot(cnt, xfull_ref[...], preferred_element_type=jnp.float32)
    deg = jnp.sum(cnt, axis=-1, keepdims=True).astype(jnp.float32)
    msg = (m * (1.0 / jnp.maximum(deg, 1.0))).astype(cnt.dtype)
    xt = xfull_ref[pl.ds(i * tm, tm), :]
    h = (jnp.dot(xt, wl_ref[...], preferred_element_type=jnp.float32)
         + jnp.dot(msg, wr_ref[...], preferred_element_type=jnp.float32))

    inv_f = 1.0 / out_dim
    s = jnp.sum(h, axis=-1, keepdims=True)
    ss = jnp.sum(h * h, axis=-1, keepdims=True)
    mean = s * inv_f
    var = ss * inv_f - mean * mean
    y = (h - mean) * jax.lax.rsqrt(var + eps) * gamma_ref[...] + beta_ref[...]
    if residual:
        y = y + xt.astype(jnp.float32)
    if relu:
        y = jnp.maximum(y, 0.0)
    out_ref[...] = y.astype(out_ref.dtype)


def _layer(cnt, x_bf, wl_t, wr_t, gamma, beta, *, out_dim, residual, relu,
           out_dtype, eps=1e-5):
    n, in_p = x_bf.shape
    out_p = wl_t.shape[1]
    tm = 512 if n % 512 == 0 else n
    body = functools.partial(_layer_kernel, tm=tm, out_dim=out_dim, eps=eps,
                             residual=residual, relu=relu)
    return pl.pallas_call(
        body,
        out_shape=jax.ShapeDtypeStruct((n, out_p), out_dtype),
        grid=(n // tm,),
        in_specs=[
            pl.BlockSpec((tm, n), lambda i: (i, 0)),
            pl.BlockSpec((n, in_p), lambda i: (0, 0)),
            pl.BlockSpec((in_p, out_p), lambda i: (0, 0)),
            pl.BlockSpec((in_p, out_p), lambda i: (0, 0)),
            pl.BlockSpec((1, out_p), lambda i: (0, 0)),
            pl.BlockSpec((1, out_p), lambda i: (0, 0)),
        ],
        out_specs=pl.BlockSpec((tm, out_p), lambda i: (i, 0)),
        compiler_params=pltpu.CompilerParams(
            dimension_semantics=("parallel",)),
    )(cnt, x_bf, wl_t, wr_t, gamma, beta)


def kernel(x, edge_index,
           l0_w_l, l0_a_l, l0_b_l, l0_w_r, l0_a_r, l0_b_r, l0_gamma, l0_beta,
           l1_w_l, l1_a_l, l1_b_l, l1_w_r, l1_a_r, l1_b_r, l1_gamma, l1_beta):
    n = x.shape[0]
    scaling = 2.0
    bf = jnp.bfloat16

    wl0 = (l0_w_l.T + scaling * (l0_a_l.T @ l0_b_l.T)).astype(bf)
    wr0 = (l0_w_r.T + scaling * (l0_a_r.T @ l0_b_r.T)).astype(bf)
    wl1 = (l1_w_l.T + scaling * (l1_a_l.T @ l1_b_l.T)).astype(bf)
    wr1 = (l1_w_r.T + scaling * (l1_a_r.T @ l1_b_r.T)).astype(bf)
    g0 = l0_gamma.reshape(1, -1).astype(jnp.float32)
    b0 = l0_beta.reshape(1, -1).astype(jnp.float32)
    g1 = l1_gamma.reshape(1, -1).astype(jnp.float32)
    b1 = l1_beta.reshape(1, -1).astype(jnp.float32)

    src, dst = edge_index[0], edge_index[1]
    tm = 512 if n % 512 == 0 else n
    (keys2, crow, ccol, chunk, first, last,
     nitems) = _edge_tables(src, dst, n, tm)
    cnt = _build_counts(keys2, crow, ccol, chunk, first, last,
                        n, tm, nitems, bf)

    hid = wl0.shape[1]
    out_d = wl1.shape[1]
    h1 = _layer(cnt, x.astype(bf), wl0, wr0, g0, b0, out_dim=hid,
                residual=True, relu=True, out_dtype=bf)
    out = _layer(cnt, h1, wl1, wr1, g1, b1, out_dim=out_d,
                 residual=False, relu=False, out_dtype=jnp.float32)
    return out


# speedup vs baseline: 1.5056x; 1.0767x over previous
import functools

import jax
import jax.numpy as jnp
from jax.experimental import pallas as pl
from jax.experimental.pallas import tpu as pltpu

_CH = 1024
_CG = 512


def _build_kernel(crow_ref, ccol_ref, chunk_ref, first_ref, last_ref,
                  keys_ref, out_ref, *, tm, n, cg):
    t = pl.program_id(0)
    cell = crow_ref[t] * (n // cg) + ccol_ref[t]
    keyv = keys_ref[0]
    hi = keyv >> 9
    cl = keyv & 511
    rl_iota = jax.lax.broadcasted_iota(jnp.int32, (tm, 1), 0)
    cl_iota = jax.lax.broadcasted_iota(jnp.int32, (cg, 1), 0)
    f8 = jnp.float8_e4m3fn
    ohr = (hi == cell * 512 + rl_iota).astype(f8)
    ohc = (cl == cl_iota).astype(f8)
    m = jax.lax.dot_general(ohr, ohc, (((1,), (1,)), ((), ())),
                            preferred_element_type=jnp.float32)

    @pl.when(first_ref[t] == 1)
    def _():
        out_ref[...] = m.astype(out_ref.dtype)

    @pl.when(first_ref[t] == 0)
    def _():
        out_ref[...] = out_ref[...] + m.astype(out_ref.dtype)


def _build_counts(keys2, crow, ccol, chunk, first, last,
                  n, tm, nitems, dtype):
    cg = min(_CG, n)
    return pl.pallas_call(
        functools.partial(_build_kernel, tm=tm, n=n, cg=cg),
        out_shape=jax.ShapeDtypeStruct((n, n), dtype),
        grid_spec=pltpu.PrefetchScalarGridSpec(
            num_scalar_prefetch=5,
            grid=(nitems,),
            in_specs=[
                pl.BlockSpec(
                    (1, 1, _CH), lambda t, cr, cc, ch, fr, la: (ch[t], 0, 0)),
            ],
            out_specs=pl.BlockSpec(
                (tm, cg), lambda t, cr, cc, ch, fr, la: (cr[t], cc[t])),
        ),
        compiler_params=pltpu.CompilerParams(
            dimension_semantics=("arbitrary",)),
    )(crow, ccol, chunk, first, last, keys2)


def _edge_tables(src, dst, n, tm):
    e = src.shape[0]
    nch = -(-e // _CH)
    cg = min(_CG, n)
    ncell = (n // tm) * (n // cg)
    r = dst.astype(jnp.int32)
    c = src.astype(jnp.int32)
    cell = (r // tm) * (n // cg) + (c // cg)
    key = (cell << 18) | ((r % tm) << 9) | (c % cg)
    keys = jnp.sort(key)
    sent = jnp.int32(1 << 28)
    keys_p = jnp.concatenate(
        [keys, jnp.full((nch * _CH - e + _CH,), sent, jnp.int32)])
    keys2 = keys_p.reshape(nch + 1, 1, _CH)

    qidx = jnp.arange(nch, dtype=jnp.int32)
    first_cell = keys_p[qidx * _CH] >> 18
    last_cell = keys[jnp.minimum((qidx + 1) * _CH - 1, e - 1)] >> 18
    cells = jnp.arange(ncell, dtype=jnp.int32)
    lo = jnp.searchsorted(last_cell, cells, side='left').astype(jnp.int32)
    hi = jnp.searchsorted(first_cell, cells, side='right').astype(jnp.int32) - 1
    cnt_c = jnp.maximum(hi - lo + 1, 1)
    cum = jnp.concatenate(
        [jnp.zeros((1,), jnp.int32), jnp.cumsum(cnt_c).astype(jnp.int32)])

    nitems = ncell + nch - 1
    tt = jnp.arange(nitems, dtype=jnp.int32)
    cell_t = jnp.clip(
        jnp.searchsorted(cum, tt, side='right').astype(jnp.int32) - 1,
        0, ncell - 1)
    k_t = tt - cum[cell_t]
    valid = k_t <= hi[cell_t] - lo[cell_t]
    chunk_t = jnp.where(valid, lo[cell_t] + k_t, nch).astype(jnp.int32)
    first_t = (k_t == 0).astype(jnp.int32)
    last_t = (k_t == cnt_c[cell_t] - 1).astype(jnp.int32)
    crow_t = (cell_t // (n // cg)).astype(jnp.int32)
    ccol_t = (cell_t % (n // cg)).astype(jnp.int32)
    return keys2, crow_t, ccol_t, chunk_t, first_t, last_t, nitems


def _layer_kernel(cnt_ref, xfull_ref, wl_ref, wr_ref, gamma_ref, beta_ref,
                  out_ref, *, tm, out_dim, eps, residual, relu):
    i = pl.program_id(0)
    cnt = cnt_ref[...]
    m = jnp.dot(cnt, xfull_ref[...], preferred_element_type=jnp.float32)
    deg = jnp.sum(cnt, axis=-1, keepdims=True).astype(jnp.float32)
    msg = (m * (1.0 / jnp.maximum(deg, 1.0))).astype(cnt.dtype)
    xt = xfull_ref[pl.ds(i * tm, tm), :]
    h = (jnp.dot(xt, wl_ref[...], preferred_element_type=jnp.float32)
         + jnp.dot(msg, wr_ref[...], preferred_element_type=jnp.float32))

    inv_f = 1.0 / out_dim
    s = jnp.sum(h, axis=-1, keepdims=True)
    ss = jnp.sum(h * h, axis=-1, keepdims=True)
    mean = s * inv_f
    var = ss * inv_f - mean * mean
    y = (h - mean) * jax.lax.rsqrt(var + eps) * gamma_ref[...] + beta_ref[...]
    if residual:
        y = y + xt.astype(jnp.float32)
    if relu:
        y = jnp.maximum(y, 0.0)
    out_ref[...] = y.astype(out_ref.dtype)


def _layer(cnt, x_bf, wl_t, wr_t, gamma, beta, *, out_dim, residual, relu,
           out_dtype, eps=1e-5):
    n, in_p = x_bf.shape
    out_p = wl_t.shape[1]
    tm = 512 if n % 512 == 0 else n
    body = functools.partial(_layer_kernel, tm=tm, out_dim=out_dim, eps=eps,
                             residual=residual, relu=relu)
    return pl.pallas_call(
        body,
        out_shape=jax.ShapeDtypeStruct((n, out_p), out_dtype),
        grid=(n // tm,),
        in_specs=[
            pl.BlockSpec((tm, n), lambda i: (i, 0)),
            pl.BlockSpec((n, in_p), lambda i: (0, 0)),
            pl.BlockSpec((in_p, out_p), lambda i: (0, 0)),
            pl.BlockSpec((in_p, out_p), lambda i: (0, 0)),
            pl.BlockSpec((1, out_p), lambda i: (0, 0)),
            pl.BlockSpec((1, out_p), lambda i: (0, 0)),
        ],
        out_specs=pl.BlockSpec((tm, out_p), lambda i: (i, 0)),
        compiler_params=pltpu.CompilerParams(
            dimension_semantics=("parallel",)),
    )(cnt, x_bf, wl_t, wr_t, gamma, beta)


def kernel(x, edge_index,
           l0_w_l, l0_a_l, l0_b_l, l0_w_r, l0_a_r, l0_b_r, l0_gamma, l0_beta,
           l1_w_l, l1_a_l, l1_b_l, l1_w_r, l1_a_r, l1_b_r, l1_gamma, l1_beta):
    n = x.shape[0]
    scaling = 2.0
    bf = jnp.bfloat16

    wl0 = (l0_w_l.T + scaling * (l0_a_l.T @ l0_b_l.T)).astype(bf)
    wr0 = (l0_w_r.T + scaling * (l0_a_r.T @ l0_b_r.T)).astype(bf)
    wl1 = (l1_w_l.T + scaling * (l1_a_l.T @ l1_b_l.T)).astype(bf)
    wr1 = (l1_w_r.T + scaling * (l1_a_r.T @ l1_b_r.T)).astype(bf)
    g0 = l0_gamma.reshape(1, -1).astype(jnp.float32)
    b0 = l0_beta.reshape(1, -1).astype(jnp.float32)
    g1 = l1_gamma.reshape(1, -1).astype(jnp.float32)
    b1 = l1_beta.reshape(1, -1).astype(jnp.float32)

    src, dst = edge_index[0], edge_index[1]
    tm = 512 if n % 512 == 0 else n
    (keys2, crow, ccol, chunk, first, last,
     nitems) = _edge_tables(src, dst, n, tm)
    cnt = _build_counts(keys2, crow, ccol, chunk, first, last,
                        n, tm, nitems, bf)

    hid = wl0.shape[1]
    out_d = wl1.shape[1]
    h1 = _layer(cnt, x.astype(bf), wl0, wr0, g0, b0, out_dim=hid,
                residual=True, relu=True, out_dtype=bf)
    out = _layer(cnt, h1, wl1, wr1, g1, b1, out_dim=out_d,
                 residual=False, relu=False, out_dtype=jnp.float32)
    return out
